# baseline (speedup 1.0000x reference)
import functools

import jax
import jax.numpy as jnp
from jax import lax
from jax.experimental import pallas as pl
from jax.experimental.pallas import tpu as pltpu
from jax.experimental.pallas import tpu_sc as plsc

_P = 105
_D = 2
_LANES = 16
_LP = 112
_MP = 224
_NEG = -1e30

_mesh = plsc.VectorSubcoreMesh(core_axis_name="c", subcore_axis_name="s")


def _body(logits_hbm, mag_hbm, probs_hbm, sig_hbm, lv, mv):
    wid = lax.axis_index("s") * 2 + lax.axis_index("c")

    @pl.when(wid == 0)
    def _softmax():
        pltpu.sync_copy(logits_hbm, lv)
        vecs = [lv[pl.ds(_LANES * i, _LANES)] for i in range(_LP // _LANES)]
        vmax = vecs[0]
        for v in vecs[1:]:
            vmax = jnp.maximum(vmax, v)
        m = jnp.max(vmax)
        exps = [jnp.exp(v - m) for v in vecs]
        vsum = exps[0]
        for e in exps[1:]:
            vsum = vsum + e
        inv = 1.0 / jnp.broadcast_to(jnp.sum(vsum), (_LANES,))
        for i, e in enumerate(exps):
            lv[pl.ds(_LANES * i, _LANES)] = e * inv
        pltpu.sync_copy(lv, probs_hbm)

    @pl.when(wid == 1)
    def _sigmoid():
        pltpu.sync_copy(mag_hbm, mv)
        for i in range(_MP // _LANES):
            x = mv[pl.ds(_LANES * i, _LANES)]
            mv[pl.ds(_LANES * i, _LANES)] = 1.0 / (1.0 + jnp.exp(-x))
        pltpu.sync_copy(mv, sig_hbm)


_sc_call = pl.kernel(
    _body,
    out_type=(
        jax.ShapeDtypeStruct((_LP,), jnp.float32),
        jax.ShapeDtypeStruct((_MP,), jnp.float32),
    ),
    mesh=_mesh,
    scratch_types=(
        pltpu.VMEM((_LP,), jnp.float32),
        pltpu.VMEM((_MP,), jnp.float32),
    ),
    name="optaug_policy_sc",
    compiler_params=pltpu.CompilerParams(needs_layout_passes=False),
)


def kernel(x, logits, mag_params):
    del x
    lp = jnp.pad(logits, (0, _LP - _P), constant_values=_NEG)
    mp = jnp.pad(mag_params.reshape(-1), (0, _MP - _P * _D))
    probs, sig = _sc_call(lp, mp)
    return probs[:_P], sig[: _P * _D].reshape(_P, _D)


# speedup vs baseline: 1.0594x; 1.0594x over previous
import jax
import jax.numpy as jnp
from jax import lax
from jax.experimental import pallas as pl
from jax.experimental.pallas import tpu as pltpu
from jax.experimental.pallas import tpu_sc as plsc

_P = 105
_D = 2
_M = _P * _D
_LANES = 16
_LVECS = 7
_MVECS = 14
_TAIL = _P - (_LVECS - 1) * _LANES
_NEG = -1e30

_mesh = plsc.VectorSubcoreMesh(
    core_axis_name="c", subcore_axis_name="s", num_cores=1
)


def _body(logits_hbm, mag_hbm, probs_hbm, sig_hbm, lv, mv):
    sid = lax.axis_index("s")

    @pl.when(sid == 0)
    def _softmax():
        pltpu.sync_copy(logits_hbm, lv.at[pl.ds(0, _P)])
        valid = lax.iota(jnp.int32, _LANES) < _TAIL
        vecs = [lv[pl.ds(_LANES * i, _LANES)] for i in range(_LVECS)]
        vecs[-1] = jnp.where(valid, vecs[-1], _NEG)
        vmax = vecs[0]
        for v in vecs[1:]:
            vmax = jnp.maximum(vmax, v)
        m = jnp.max(vmax)
        exps = [jnp.exp(v - m) for v in vecs]
        vsum = exps[0]
        for e in exps[1:]:
            vsum = vsum + e
        inv = 1.0 / jnp.broadcast_to(jnp.sum(vsum), (_LANES,))
        for i, e in enumerate(exps):
            lv[pl.ds(_LANES * i, _LANES)] = e * inv
        pltpu.sync_copy(lv.at[pl.ds(0, _P)], probs_hbm)

    @pl.when(sid == 1)
    def _sigmoid():
        pltpu.sync_copy(mag_hbm, mv.at[pl.ds(0, _M)])
        for i in range(_MVECS):
            x = mv[pl.ds(_LANES * i, _LANES)]
            mv[pl.ds(_LANES * i, _LANES)] = 1.0 / (1.0 + jnp.exp(-x))
        pltpu.sync_copy(mv.at[pl.ds(0, _M)], sig_hbm)


_sc_call = pl.kernel(
    _body,
    out_type=(
        jax.ShapeDtypeStruct((_P,), jnp.float32),
        jax.ShapeDtypeStruct((_M,), jnp.float32),
    ),
    mesh=_mesh,
    scratch_types=(
        pltpu.VMEM((_LVECS * _LANES,), jnp.float32),
        pltpu.VMEM((_MVECS * _LANES,), jnp.float32),
    ),
    name="optaug_policy_sc",
    compiler_params=pltpu.CompilerParams(needs_layout_passes=False),
)


def kernel(x, logits, mag_params):
    del x
    probs, sig = _sc_call(logits, mag_params.reshape(_M))
    return probs, sig.reshape(_P, _D)
